# baseline (speedup 1.0000x reference)
import jax
import jax.numpy as jnp
from jax.experimental import pallas as pl
from jax.experimental.pallas import tpu as pltpu

MARGIN = 0.3
PD_EPS = 1e-6
B = 1024
D = 2048
BM = 256
NI = B // BM


def _triplet_kernel(a_ref, p_ref, out_ref, pn_ref, colp_ref):
    i = pl.program_id(0)

    @pl.when(i == 0)
    def _init():
        p = p_ref[...]
        nrm = jnp.sqrt(jnp.sum(p * p, axis=1, keepdims=True))
        pn = p / jnp.maximum(nrm, 1e-12)
        pn_ref[...] = pn.astype(jnp.bfloat16)
        colp_ref[...] = (jnp.sum(pn * pn, axis=1, keepdims=True)
                         - (2.0 * PD_EPS) * jnp.sum(pn, axis=1, keepdims=True))
        out_ref[...] = jnp.zeros_like(out_ref)

    a = a_ref[...]
    nrma = jnp.sqrt(jnp.sum(a * a, axis=1, keepdims=True))
    an = (a / jnp.maximum(nrma, 1e-12)).astype(jnp.bfloat16)

    cross = jax.lax.dot_general(
        pn_ref[...], an, (((1,), (1,)), ((), ())),
        preferred_element_type=jnp.float32)
    h = colp_ref[...] - 2.0 * cross

    rowj = jax.lax.broadcasted_iota(jnp.int32, (B, BM), 0)
    coli = jax.lax.broadcasted_iota(jnp.int32, (B, BM), 1) + i * BM
    diag = rowj == coli

    hneg = jnp.min(jnp.where(diag, jnp.float32(3.0e38), h), axis=0,
                   keepdims=True)
    hpos = jnp.sum(jnp.where(diag, h, 0.0), axis=0, keepdims=True)
    lv = jnp.maximum(hpos - hneg + MARGIN, 0.0) * (1.0 / B)
    out_ref[...] += jnp.sum(lv, axis=1, keepdims=True)


def kernel(x1, x2):
    a2 = x1.reshape(B, 2 * D)
    p2 = x2.reshape(B, 2 * D)
    out = pl.pallas_call(
        _triplet_kernel,
        grid=(NI,),
        in_specs=[
            pl.BlockSpec((BM, D), lambda i: (i, 0)),
            pl.BlockSpec((B, D), lambda i: (0, 1)),
        ],
        out_specs=pl.BlockSpec((1, 1), lambda i: (0, 0)),
        out_shape=jax.ShapeDtypeStruct((1, 1), jnp.float32),
        scratch_shapes=[
            pltpu.VMEM((B, D), jnp.bfloat16),
            pltpu.VMEM((B, 1), jnp.float32),
        ],
        compiler_params=pltpu.CompilerParams(
            dimension_semantics=("arbitrary",),
        ),
    )(a2, p2)
    return out[0, 0]


# speedup vs baseline: 1.5344x; 1.5344x over previous
import jax
import jax.numpy as jnp
from jax.experimental import pallas as pl
from jax.experimental.pallas import tpu as pltpu

MARGIN = 0.3
PD_EPS = 1e-6
B = 1024
D = 2048
BM = 256
NI = B // BM


def _triplet_kernel(a_ref, p_ref, out_ref, pn_ref, colp_ref):
    i = pl.program_id(0)

    @pl.when(i == 0)
    def _init():
        PC = 256
        for c in range(B // PC):
            p = p_ref[c * PC:(c + 1) * PC, 1, :]
            nrm = jnp.sqrt(jnp.sum(p * p, axis=1, keepdims=True))
            pn = p / jnp.maximum(nrm, 1e-12)
            pn_ref[c * PC:(c + 1) * PC, :] = pn.astype(jnp.bfloat16)
            colp_ref[c * PC:(c + 1) * PC, :] = (
                jnp.sum(pn * pn, axis=1, keepdims=True)
                - (2.0 * PD_EPS) * jnp.sum(pn, axis=1, keepdims=True))
        out_ref[...] = jnp.zeros_like(out_ref)

    a = a_ref[:, 0, :]
    nrma = jnp.sqrt(jnp.sum(a * a, axis=1, keepdims=True))
    an = (a / jnp.maximum(nrma, 1e-12)).astype(jnp.bfloat16)

    cross = jax.lax.dot_general(
        pn_ref[...], an, (((1,), (1,)), ((), ())),
        preferred_element_type=jnp.float32)
    h = colp_ref[...] - 2.0 * cross

    rowj = jax.lax.broadcasted_iota(jnp.int32, (B, BM), 0)
    coli = jax.lax.broadcasted_iota(jnp.int32, (B, BM), 1) + i * BM
    diag = rowj == coli

    hneg = jnp.min(jnp.where(diag, jnp.float32(3.0e38), h), axis=0,
                   keepdims=True)
    hpos = jnp.sum(jnp.where(diag, h, 0.0), axis=0, keepdims=True)
    lv = jnp.maximum(hpos - hneg + MARGIN, 0.0) * (1.0 / B)
    out_ref[...] += jnp.sum(lv, axis=1, keepdims=True)


def kernel(x1, x2):
    out = pl.pallas_call(
        _triplet_kernel,
        grid=(NI,),
        in_specs=[
            pl.BlockSpec((BM, 2, D), lambda i: (i, 0, 0)),
            pl.BlockSpec((B, 2, D), lambda i: (0, 0, 0)),
        ],
        out_specs=pl.BlockSpec((1, 1), lambda i: (0, 0)),
        out_shape=jax.ShapeDtypeStruct((1, 1), jnp.float32),
        scratch_shapes=[
            pltpu.VMEM((B, D), jnp.bfloat16),
            pltpu.VMEM((B, 1), jnp.float32),
        ],
        compiler_params=pltpu.CompilerParams(
            dimension_semantics=("arbitrary",),
        ),
    )(x1, x2)
    return out[0, 0]


# speedup vs baseline: 2.0090x; 1.3093x over previous
import jax
import jax.numpy as jnp
from jax.experimental import pallas as pl
from jax.experimental.pallas import tpu as pltpu

MARGIN = 0.3
PD_EPS = 1e-6
B = 1024
D = 2048
BM = 256
NI = B // BM
PC = 256
NC = B // PC


def _triplet_kernel(x1_ref, x2_ref, out_ref,
                    pbf_ref, t2_ref, colp_ref,
                    araw_ref, praw_ref, asem, psem):
    i = pl.program_id(0)
    ones_row = jnp.ones((1, D), jnp.float32)

    @pl.when(i == 0)
    def _init():
        for c in range(NC):
            pltpu.make_async_copy(
                x2_ref.at[pl.ds(c * PC, PC), pl.ds(1, 1), :],
                praw_ref.at[c], psem.at[c]).start()
        pltpu.make_async_copy(
            x1_ref.at[pl.ds(0, BM), pl.ds(0, 1), :],
            araw_ref.at[0], asem.at[0]).start()
        pltpu.make_async_copy(
            x1_ref.at[pl.ds(BM, BM), pl.ds(0, 1), :],
            araw_ref.at[1], asem.at[1]).start()
        out_ref[...] = jnp.zeros_like(out_ref)
        for c in range(NC):
            pltpu.make_async_copy(
                x2_ref.at[pl.ds(c * PC, PC), pl.ds(1, 1), :],
                praw_ref.at[c], psem.at[c]).wait()
            praw = praw_ref[c, :, 0, :]
            pbf_ref[c * PC:(c + 1) * PC, :] = praw.astype(jnp.bfloat16)
            np2 = jax.lax.dot_general(
                praw * praw, ones_row, (((1,), (1,)), ((), ())),
                preferred_element_type=jnp.float32)
            sump = jax.lax.dot_general(
                praw, ones_row, (((1,), (1,)), ((), ())),
                preferred_element_type=jnp.float32)
            t = 1.0 / jnp.maximum(jnp.sqrt(np2), 1e-12)
            t2_ref[c * PC:(c + 1) * PC, :] = 2.0 * t
            colp_ref[c * PC:(c + 1) * PC, :] = np2 * t * t - (2.0 * PD_EPS) * sump * t

    @pl.when(jnp.logical_and(i >= 1, i < NI - 1))
    def _prefetch():
        pltpu.make_async_copy(
            x1_ref.at[pl.ds((i + 1) * BM, BM), pl.ds(0, 1), :],
            araw_ref.at[(i + 1) % 2], asem.at[(i + 1) % 2]).start()

    pltpu.make_async_copy(
        x1_ref.at[pl.ds(i * BM, BM), pl.ds(0, 1), :],
        araw_ref.at[i % 2], asem.at[i % 2]).wait()
    a = araw_ref[i % 2, :, 0, :]
    abf = a.astype(jnp.bfloat16)
    na2 = jax.lax.dot_general(
        ones_row, a * a, (((1,), (1,)), ((), ())),
        preferred_element_type=jnp.float32)
    sa = 1.0 / jnp.maximum(jnp.sqrt(na2), 1e-12)

    cross = jax.lax.dot_general(
        pbf_ref[...], abf, (((1,), (1,)), ((), ())),
        preferred_element_type=jnp.float32)
    h = colp_ref[...] - (t2_ref[...] * cross) * sa

    rowj = jax.lax.broadcasted_iota(jnp.int32, (B, BM), 0)
    coli = jax.lax.broadcasted_iota(jnp.int32, (B, BM), 1) + i * BM
    diag = rowj == coli

    hneg = jnp.min(jnp.where(diag, jnp.float32(3.0e38), h), axis=0,
                   keepdims=True)
    hpos = jnp.sum(jnp.where(diag, h, 0.0), axis=0, keepdims=True)
    lv = jnp.maximum(hpos - hneg + MARGIN, 0.0) * (1.0 / B)
    out_ref[...] += jnp.sum(lv, axis=1, keepdims=True)


def kernel(x1, x2):
    out = pl.pallas_call(
        _triplet_kernel,
        grid=(NI,),
        in_specs=[
            pl.BlockSpec(memory_space=pltpu.HBM),
            pl.BlockSpec(memory_space=pltpu.HBM),
        ],
        out_specs=pl.BlockSpec((1, 1), lambda i: (0, 0)),
        out_shape=jax.ShapeDtypeStruct((1, 1), jnp.float32),
        scratch_shapes=[
            pltpu.VMEM((B, D), jnp.bfloat16),
            pltpu.VMEM((B, 1), jnp.float32),
            pltpu.VMEM((B, 1), jnp.float32),
            pltpu.VMEM((2, BM, 1, D), jnp.float32),
            pltpu.VMEM((NC, PC, 1, D), jnp.float32),
            pltpu.SemaphoreType.DMA((2,)),
            pltpu.SemaphoreType.DMA((NC,)),
        ],
        compiler_params=pltpu.CompilerParams(
            dimension_semantics=("arbitrary",),
        ),
    )(x1, x2)
    return out[0, 0]


# speedup vs baseline: 5.2441x; 2.6103x over previous
import jax
import jax.numpy as jnp
from jax.experimental import pallas as pl
from jax.experimental.pallas import tpu as pltpu

MARGIN = 0.3
PD_EPS = 1e-6
B = 1024
D = 2048
BM = 256
NI = B // BM
PC = 256
NC = B // PC


def _triplet_kernel(x1_ref, x2_ref, out_ref,
                    pbf_ref, t2_ref, colp_ref,
                    araw_ref, praw_ref, asem, psem):
    i = pl.program_id(0)
    ones_row = jnp.ones((1, D), jnp.float32)

    @pl.when(i == 0)
    def _init():
        for c in range(NC):
            pltpu.make_async_copy(
                x2_ref.at[pl.ds(c * PC, PC), 1, :],
                praw_ref.at[c], psem.at[c]).start()
        pltpu.make_async_copy(
            x1_ref.at[pl.ds(0, BM), 0, :],
            araw_ref.at[0], asem.at[0]).start()
        pltpu.make_async_copy(
            x1_ref.at[pl.ds(BM, BM), 0, :],
            araw_ref.at[1], asem.at[1]).start()
        out_ref[...] = jnp.zeros_like(out_ref)
        for c in range(NC):
            pltpu.make_async_copy(
                x2_ref.at[pl.ds(c * PC, PC), 1, :],
                praw_ref.at[c], psem.at[c]).wait()
            praw = praw_ref[c]
            pbf_ref[c * PC:(c + 1) * PC, :] = praw.astype(jnp.bfloat16)
            np2 = jax.lax.dot_general(
                praw * praw, ones_row, (((1,), (1,)), ((), ())),
                preferred_element_type=jnp.float32)
            sump = jax.lax.dot_general(
                praw, ones_row, (((1,), (1,)), ((), ())),
                preferred_element_type=jnp.float32)
            t = 1.0 / jnp.maximum(jnp.sqrt(np2), 1e-12)
            t2_ref[c * PC:(c + 1) * PC, :] = 2.0 * t
            colp_ref[c * PC:(c + 1) * PC, :] = np2 * t * t - (2.0 * PD_EPS) * sump * t

    @pl.when(jnp.logical_and(i >= 1, i < NI - 1))
    def _prefetch():
        pltpu.make_async_copy(
            x1_ref.at[pl.ds((i + 1) * BM, BM), 0, :],
            araw_ref.at[(i + 1) % 2], asem.at[(i + 1) % 2]).start()

    pltpu.make_async_copy(
        x1_ref.at[pl.ds(i * BM, BM), 0, :],
        araw_ref.at[i % 2], asem.at[i % 2]).wait()
    a = araw_ref[i % 2]
    abf = a.astype(jnp.bfloat16)
    na2 = jax.lax.dot_general(
        ones_row, a * a, (((1,), (1,)), ((), ())),
        preferred_element_type=jnp.float32)
    sa = 1.0 / jnp.maximum(jnp.sqrt(na2), 1e-12)

    cross = jax.lax.dot_general(
        pbf_ref[...], abf, (((1,), (1,)), ((), ())),
        preferred_element_type=jnp.float32)
    h = colp_ref[...] - (t2_ref[...] * cross) * sa

    rowj = jax.lax.broadcasted_iota(jnp.int32, (B, BM), 0)
    coli = jax.lax.broadcasted_iota(jnp.int32, (B, BM), 1) + i * BM
    diag = rowj == coli

    hneg = jnp.min(jnp.where(diag, jnp.float32(3.0e38), h), axis=0,
                   keepdims=True)
    hpos = jnp.sum(jnp.where(diag, h, 0.0), axis=0, keepdims=True)
    lv = jnp.maximum(hpos - hneg + MARGIN, 0.0) * (1.0 / B)
    out_ref[...] += jnp.sum(lv, axis=1, keepdims=True)


def kernel(x1, x2):
    out = pl.pallas_call(
        _triplet_kernel,
        grid=(NI,),
        in_specs=[
            pl.BlockSpec(memory_space=pltpu.HBM),
            pl.BlockSpec(memory_space=pltpu.HBM),
        ],
        out_specs=pl.BlockSpec((1, 1), lambda i: (0, 0)),
        out_shape=jax.ShapeDtypeStruct((1, 1), jnp.float32),
        scratch_shapes=[
            pltpu.VMEM((B, D), jnp.bfloat16),
            pltpu.VMEM((B, 1), jnp.float32),
            pltpu.VMEM((B, 1), jnp.float32),
            pltpu.VMEM((2, BM, D), jnp.float32),
            pltpu.VMEM((NC, PC, D), jnp.float32),
            pltpu.SemaphoreType.DMA((2,)),
            pltpu.SemaphoreType.DMA((NC,)),
        ],
        compiler_params=pltpu.CompilerParams(
            dimension_semantics=("arbitrary",),
        ),
    )(x1, x2)
    return out[0, 0]
